# Initial kernel scaffold; baseline (speedup 1.0000x reference)
import jax
import jax.numpy as jnp
from jax.experimental import pallas as pl


def kernel(x, op_id, expert_key, sW1, sb1, sW2, sb2, eW1, eb1, eW2, eb2, gate_logit):
    raise NotImplementedError("write your pallas kernel here")


# speedup vs baseline: 1.3518x; 1.3518x over previous
import functools

import jax
import jax.numpy as jnp
from jax import lax
from jax.experimental import pallas as pl
from jax.experimental.pallas import tpu as pltpu

T, C, H = 2048, 768, 3072
NB, EPB = 4, 2
E = NB * EPB
TAU = 1.0
BLK = 128
G = T // BLK + NB
T_PAD = G * BLK


def _router_body(x_ref, ke_ref, ko_ref, ids_ref, gate_ref, p1_ref, alpha_ref):
    ke = ke_ref[:]
    ko = ko_ref[:]
    ke = ke / jnp.maximum(jnp.sqrt(jnp.sum(ke * ke, axis=1, keepdims=True)), 1e-12)
    ko = ko / jnp.maximum(jnp.sqrt(jnp.sum(ko * ko, axis=1, keepdims=True)), 1e-12)
    d = ko - ke
    xv = x_ref[:]
    nrm = jnp.maximum(jnp.sqrt(jnp.sum(xv * xv, axis=1, keepdims=True)), 1e-12)
    z4 = lax.dot_general(xv, d, (((1,), (1,)), ((), ())),
                         preferred_element_type=jnp.float32)
    z4 = z4 / nrm / max(TAU, 1e-6)
    ids = jnp.clip(ids_ref[:], 0, NB - 1)
    z = jnp.zeros_like(nrm)
    for b in range(NB):
        z = jnp.where(ids == b, z4[:, b:b + 1], z)
    p1_ref[:] = jax.nn.sigmoid(z)
    alpha_ref[:, :] = jax.nn.sigmoid(gate_ref[:, :])


def _router(x2, keys_e, keys_o, ids2, gate):
    return pl.pallas_call(
        _router_body,
        out_shape=[
            jax.ShapeDtypeStruct((T, 1), jnp.float32),
            jax.ShapeDtypeStruct((1, 1), jnp.float32),
        ],
    )(x2, keys_e, keys_o, ids2, gate)


def _moe_body(tb_ref, xs_ref, p1_ref, w1_ref, b1_ref, w2_ref, b2_ref,
              alpha_ref, out_ref):
    xv = xs_ref[:].astype(jnp.bfloat16)
    p1 = p1_ref[:]
    h0 = jnp.maximum(
        jnp.dot(xv, w1_ref[0, 0], preferred_element_type=jnp.float32)
        + b1_ref[0, 0], 0.0).astype(jnp.bfloat16)
    h1 = jnp.maximum(
        jnp.dot(xv, w1_ref[0, 1], preferred_element_type=jnp.float32)
        + b1_ref[0, 1], 0.0).astype(jnp.bfloat16)
    y0 = jnp.dot(h0, w2_ref[0, 0], preferred_element_type=jnp.float32) + b2_ref[0, 0]
    y1 = jnp.dot(h1, w2_ref[0, 1], preferred_element_type=jnp.float32) + b2_ref[0, 1]
    out_ref[:] = alpha_ref[0, 0] * ((1.0 - p1) * y0 + p1 * y1)


def _moe_ffn(tb, xs, p1s, w1p, b1p, w2p, b2p, alpha):
    grid_spec = pltpu.PrefetchScalarGridSpec(
        num_scalar_prefetch=1,
        grid=(G,),
        in_specs=[
            pl.BlockSpec((BLK, C), lambda i, tb: (i, 0)),
            pl.BlockSpec((BLK, 1), lambda i, tb: (i, 0)),
            pl.BlockSpec((1, EPB, C, H), lambda i, tb: (tb[i], 0, 0, 0)),
            pl.BlockSpec((1, EPB, H), lambda i, tb: (tb[i], 0, 0)),
            pl.BlockSpec((1, EPB, H, C), lambda i, tb: (tb[i], 0, 0, 0)),
            pl.BlockSpec((1, EPB, C), lambda i, tb: (tb[i], 0, 0)),
            pl.BlockSpec(memory_space=pltpu.SMEM),
        ],
        out_specs=pl.BlockSpec((BLK, C), lambda i, tb: (i, 0)),
    )
    return pl.pallas_call(
        _moe_body,
        grid_spec=grid_spec,
        out_shape=jax.ShapeDtypeStruct((T_PAD, C), jnp.float32),
    )(tb, xs, p1s, w1p, b1p, w2p, b2p, alpha)


DBLK = 256


def _dense_body(x_ref, w1_ref, b1_ref, w2_ref, b2_ref, out_ref):
    h = jnp.maximum(
        jnp.dot(x_ref[:].astype(jnp.bfloat16), w1_ref[:],
                preferred_element_type=jnp.float32)
        + b1_ref[:], 0.0).astype(jnp.bfloat16)
    out_ref[:] = (jnp.dot(h, w2_ref[:], preferred_element_type=jnp.float32)
                  + b2_ref[:])


def _dense_ffn(x2, sW1, sb1, sW2, sb2):
    return pl.pallas_call(
        _dense_body,
        grid=(T // DBLK,),
        in_specs=[
            pl.BlockSpec((DBLK, C), lambda i: (i, 0)),
            pl.BlockSpec((C, H), lambda i: (0, 0)),
            pl.BlockSpec((1, H), lambda i: (0, 0)),
            pl.BlockSpec((H, C), lambda i: (0, 0)),
            pl.BlockSpec((1, C), lambda i: (0, 0)),
        ],
        out_specs=pl.BlockSpec((DBLK, C), lambda i: (i, 0)),
        out_shape=jax.ShapeDtypeStruct((T, C), jnp.float32),
    )(x2, sW1, sb1, sW2, sb2)


def kernel(x, op_id, expert_key, sW1, sb1, sW2, sb2, eW1, eb1, eW2, eb2,
           gate_logit):
    x2 = x.reshape(T, C)
    ids = jnp.clip(op_id.reshape(T).astype(jnp.int32), 0, NB - 1)
    keys = expert_key.reshape(E, C)
    keys_e = keys[0::2]
    keys_o = keys[1::2]
    gate = gate_logit.reshape(1, 1)

    p1, alpha = _router(x2, keys_e, keys_o, ids.reshape(T, 1), gate)

    onehot = (ids[:, None] == jnp.arange(NB)[None, :]).astype(jnp.int32)
    counts = jnp.sum(onehot, axis=0)
    padded = ((counts + BLK - 1) // BLK) * BLK
    ends = jnp.cumsum(padded)
    starts = ends - padded
    cum_counts = jnp.cumsum(counts) - counts
    order = jnp.argsort(ids, stable=True)
    b_of = ids[order]
    slot = starts[b_of] + (jnp.arange(T) - cum_counts[b_of])
    perm = jnp.zeros((T_PAD,), jnp.int32).at[slot].set(order.astype(jnp.int32))
    inv = jnp.zeros((T,), jnp.int32).at[order].set(slot.astype(jnp.int32))
    tile_lo = jnp.arange(G, dtype=jnp.int32) * BLK
    tb = jnp.minimum(jnp.sum(tile_lo[:, None] >= ends[None, :], axis=1),
                     NB - 1).astype(jnp.int32)
    xs = x2[perm]
    p1s = p1[:, 0][perm].reshape(T_PAD, 1)

    w1p = eW1.reshape(NB, EPB, C, H).astype(jnp.bfloat16)
    b1p = eb1.reshape(NB, EPB, H)
    w2p = eW2.reshape(NB, EPB, H, C).astype(jnp.bfloat16)
    b2p = eb2.reshape(NB, EPB, C)

    m = _moe_ffn(tb, xs, p1s, w1p, b1p, w2p, b2p, alpha)
    dense = _dense_ffn(x2, sW1.astype(jnp.bfloat16), sb1.reshape(1, H),
                       sW2.astype(jnp.bfloat16), sb2.reshape(1, C))

    out = dense + m[inv]
    return out.reshape(x.shape)


# speedup vs baseline: 1.6692x; 1.2348x over previous
import functools

import jax
import jax.numpy as jnp
from jax import lax
from jax.experimental import pallas as pl
from jax.experimental.pallas import tpu as pltpu
from jax.experimental.pallas import tpu_sc as plsc

T, C, H = 2048, 768, 3072
NB, EPB = 4, 2
E = NB * EPB
TAU = 1.0
BLK = 128
G = T // BLK + NB
T_PAD = G * BLK

NC, NS, LANES = 2, 16, 16
NW = NC * NS
RPW = T_PAD // NW
TPW = T // NW


def _sc_sort_body(ids_hbm, x_hbm, xs_hbm, inv_hbm, tb_hbm,
                  ids_v, perm_v, inv_v, tbv, myperm, rows_v, perm_sh, sem):
    cid = lax.axis_index("c")
    sid = lax.axis_index("s")
    wid = cid * NS + sid
    l16 = lax.iota(jnp.int32, 16)

    @pl.when(sid == 0)
    def _():
        pltpu.sync_copy(ids_hbm, ids_v)

        def hist_body(i, c):
            v = ids_v[pl.ds(i * LANES, LANES)]
            return (c[0] + jnp.sum((v == 0).astype(jnp.int32)),
                    c[1] + jnp.sum((v == 1).astype(jnp.int32)),
                    c[2] + jnp.sum((v == 2).astype(jnp.int32)),
                    c[3] + jnp.sum((v == 3).astype(jnp.int32)))

        z = jnp.int32(0)
        c0, c1, c2, c3 = lax.fori_loop(0, T // LANES, hist_body, (z, z, z, z))
        p0 = ((c0 + BLK - 1) >> 7) << 7
        p1 = ((c1 + BLK - 1) >> 7) << 7
        p2 = ((c2 + BLK - 1) >> 7) << 7
        p3 = ((c3 + BLK - 1) >> 7) << 7
        e0 = p0
        e1 = e0 + p1
        e2 = e1 + p2
        e3 = e2 + p3

        for g16 in range(2):
            lo = (l16 + g16 * LANES) * BLK
            tbx = ((lo >= e0).astype(jnp.int32) + (lo >= e1).astype(jnp.int32)
                   + (lo >= e2).astype(jnp.int32) + (lo >= e3).astype(jnp.int32))
            tbv[pl.ds(g16 * LANES, LANES)] = jnp.minimum(tbx, NB - 1)

        zv = jnp.zeros((LANES,), jnp.int32)

        def zero_body(i, _):
            perm_v[pl.ds(i * LANES, LANES)] = zv
            return 0

        lax.fori_loop(0, T_PAD // LANES, zero_body, 0)

        def scat_body(i, cur):
            v = ids_v[pl.ds(i * LANES, LANES)]
            tok = l16 + i * LANES
            m0 = (v == 0).astype(jnp.int32)
            m1 = (v == 1).astype(jnp.int32)
            m2 = (v == 2).astype(jnp.int32)
            m3 = (v == 3).astype(jnp.int32)
            base = (m0 * cur[0] + m1 * cur[1] + m2 * cur[2] + m3 * cur[3])
            rank = (m0 * (jnp.cumsum(m0) - 1) + m1 * (jnp.cumsum(m1) - 1)
                    + m2 * (jnp.cumsum(m2) - 1) + m3 * (jnp.cumsum(m3) - 1))
            slot = base + rank
            plsc.store_scatter(perm_v, [slot], tok)
            inv_v[pl.ds(i * LANES, LANES)] = slot
            return (cur[0] + jnp.sum(m0), cur[1] + jnp.sum(m1),
                    cur[2] + jnp.sum(m2), cur[3] + jnp.sum(m3))

        lax.fori_loop(0, T // LANES, scat_body, (z, e0, e1, e2))

        pltpu.sync_copy(perm_v, perm_sh)

        @pl.when(cid == 0)
        def _():
            pltpu.sync_copy(inv_v, inv_hbm)
            pltpu.sync_copy(tbv, tb_hbm)

    plsc.subcore_barrier()

    base = wid * RPW
    pltpu.sync_copy(perm_sh.at[pl.ds(base, RPW)], myperm)
    pltpu.async_copy(x_hbm.at[myperm], rows_v, sem).wait()
    pltpu.sync_copy(rows_v, xs_hbm.at[pl.ds(base, RPW)])


def _sc_sort_gather(ids, x2):
    mesh = plsc.VectorSubcoreMesh(core_axis_name="c", subcore_axis_name="s")
    return pl.kernel(
        _sc_sort_body,
        out_type=[
            jax.ShapeDtypeStruct((T_PAD, C), jnp.float32),
            jax.ShapeDtypeStruct((T,), jnp.int32),
            jax.ShapeDtypeStruct((NW,), jnp.int32),
        ],
        mesh=mesh,
        compiler_params=pltpu.CompilerParams(needs_layout_passes=False),
        scratch_types=[
            pltpu.VMEM((T,), jnp.int32),
            pltpu.VMEM((T_PAD,), jnp.int32),
            pltpu.VMEM((T,), jnp.int32),
            pltpu.VMEM((NW,), jnp.int32),
            pltpu.VMEM((RPW,), jnp.int32),
            pltpu.VMEM((RPW, C), jnp.float32),
            pltpu.VMEM_SHARED((T_PAD,), jnp.int32),
            pltpu.SemaphoreType.DMA,
        ],
    )(ids, x2)


def _sc_unsort_body(os_hbm, inv_hbm, out_hbm, inv_v, rows_v, sem):
    cid = lax.axis_index("c")
    sid = lax.axis_index("s")
    wid = cid * NS + sid
    base = wid * TPW
    pltpu.sync_copy(inv_hbm.at[pl.ds(base, TPW)], inv_v)
    pltpu.async_copy(os_hbm.at[inv_v], rows_v, sem).wait()
    pltpu.sync_copy(rows_v, out_hbm.at[pl.ds(base, TPW)])


def _sc_unsort(os, inv):
    mesh = plsc.VectorSubcoreMesh(core_axis_name="c", subcore_axis_name="s")
    return pl.kernel(
        _sc_unsort_body,
        out_type=jax.ShapeDtypeStruct((T, C), jnp.float32),
        mesh=mesh,
        compiler_params=pltpu.CompilerParams(needs_layout_passes=False),
        scratch_types=[
            pltpu.VMEM((TPW,), jnp.int32),
            pltpu.VMEM((TPW, C), jnp.float32),
            pltpu.SemaphoreType.DMA,
        ],
    )(os, inv)


def _moe_body(tb_ref, xs_ref, kp_ref, w1_ref, b1_ref, w2_ref, b2_ref,
              sw1_ref, sb1_ref, sw2_ref, sb2_ref, gate_ref, out_ref):
    xf = xs_ref[:]
    xv = xf.astype(jnp.bfloat16)
    kp = kp_ref[0]
    kn = kp / jnp.maximum(jnp.sqrt(jnp.sum(kp * kp, axis=1, keepdims=True)),
                          1e-12)
    d = (kn[1:2, :] - kn[0:1, :])
    nrm = jnp.maximum(jnp.sqrt(jnp.sum(xf * xf, axis=1, keepdims=True)), 1e-12)
    z = lax.dot_general(xf, d, (((1,), (1,)), ((), ())),
                        preferred_element_type=jnp.float32)
    p1 = jax.nn.sigmoid(z / nrm / max(TAU, 1e-6))
    alpha = jax.nn.sigmoid(gate_ref[0])
    h0 = jnp.maximum(
        jnp.dot(xv, w1_ref[0, 0], preferred_element_type=jnp.float32)
        + b1_ref[0, 0], 0.0).astype(jnp.bfloat16)
    h1 = jnp.maximum(
        jnp.dot(xv, w1_ref[0, 1], preferred_element_type=jnp.float32)
        + b1_ref[0, 1], 0.0).astype(jnp.bfloat16)
    y0 = jnp.dot(h0, w2_ref[0, 0], preferred_element_type=jnp.float32) + b2_ref[0, 0]
    y1 = jnp.dot(h1, w2_ref[0, 1], preferred_element_type=jnp.float32) + b2_ref[0, 1]
    hs = jnp.maximum(
        jnp.dot(xv, sw1_ref[:], preferred_element_type=jnp.float32)
        + sb1_ref[:], 0.0).astype(jnp.bfloat16)
    dense = jnp.dot(hs, sw2_ref[:], preferred_element_type=jnp.float32) + sb2_ref[:]
    out_ref[:] = dense + alpha * ((1.0 - p1) * y0 + p1 * y1)


def _moe_ffn(tb, xs, kpair, w1p, b1p, w2p, b2p, sW1, sb1, sW2, sb2, gate):
    grid_spec = pltpu.PrefetchScalarGridSpec(
        num_scalar_prefetch=1,
        grid=(G,),
        in_specs=[
            pl.BlockSpec((BLK, C), lambda i, tb: (i, 0)),
            pl.BlockSpec((1, EPB, C), lambda i, tb: (tb[i], 0, 0)),
            pl.BlockSpec((1, EPB, C, H), lambda i, tb: (tb[i], 0, 0, 0)),
            pl.BlockSpec((1, EPB, H), lambda i, tb: (tb[i], 0, 0)),
            pl.BlockSpec((1, EPB, H, C), lambda i, tb: (tb[i], 0, 0, 0)),
            pl.BlockSpec((1, EPB, C), lambda i, tb: (tb[i], 0, 0)),
            pl.BlockSpec((C, H), lambda i, tb: (0, 0)),
            pl.BlockSpec((1, H), lambda i, tb: (0, 0)),
            pl.BlockSpec((H, C), lambda i, tb: (0, 0)),
            pl.BlockSpec((1, C), lambda i, tb: (0, 0)),
            pl.BlockSpec(memory_space=pltpu.SMEM),
        ],
        out_specs=pl.BlockSpec((BLK, C), lambda i, tb: (i, 0)),
    )
    return pl.pallas_call(
        _moe_body,
        grid_spec=grid_spec,
        out_shape=jax.ShapeDtypeStruct((T_PAD, C), jnp.float32),
    )(tb, xs, kpair, w1p, b1p, w2p, b2p, sW1, sb1, sW2, sb2, gate)


def kernel(x, op_id, expert_key, sW1, sb1, sW2, sb2, eW1, eb1, eW2, eb2,
           gate_logit):
    x2 = x.reshape(T, C)
    ids = jnp.clip(op_id.reshape(T).astype(jnp.int32), 0, NB - 1)
    kpair = expert_key.reshape(NB, EPB, C)
    gate = gate_logit.reshape(1)

    xs, inv, tb = _sc_sort_gather(ids, x2)

    w1p = eW1.reshape(NB, EPB, C, H).astype(jnp.bfloat16)
    b1p = eb1.reshape(NB, EPB, H)
    w2p = eW2.reshape(NB, EPB, H, C).astype(jnp.bfloat16)
    b2p = eb2.reshape(NB, EPB, C)

    os = _moe_ffn(tb, xs, kpair, w1p, b1p, w2p, b2p,
                  sW1.astype(jnp.bfloat16), sb1.reshape(1, H),
                  sW2.astype(jnp.bfloat16), sb2.reshape(1, C), gate)

    out = _sc_unsort(os, inv)
    return out.reshape(x.shape)


# speedup vs baseline: 1.8873x; 1.1306x over previous
import functools

import jax
import jax.numpy as jnp
from jax import lax
from jax.experimental import pallas as pl
from jax.experimental.pallas import tpu as pltpu
from jax.experimental.pallas import tpu_sc as plsc

T, C, H = 2048, 768, 3072
NB, EPB = 4, 2
E = NB * EPB
TAU = 1.0
BLK = 128
G = T // BLK + NB
T_PAD = G * BLK

NC, NS, LANES = 2, 16, 16
NW = NC * NS
RPW = T_PAD // NW
TPW = T // NW


NSEG = NW
SEG = T // NSEG


def _sc_sort_body(ids_hbm, x_hbm, xs_hbm, inv_hbm, tb_hbm,
                  ids_v, cnt2, tab_v, slot_v, tbv, rows_v, cnt_sh, sem):
    cid = lax.axis_index("c")
    sid = lax.axis_index("s")
    g = cid * NS + sid
    l16 = lax.iota(jnp.int32, 16)
    z = jnp.int32(0)

    pltpu.sync_copy(ids_hbm.at[pl.ds(sid * 2 * SEG, 2 * SEG)], ids_v)
    for seg in range(2):
        cnt = (z, z, z, z)
        for ch in range(SEG // LANES):
            v = ids_v[pl.ds((seg * SEG // LANES + ch) * LANES, LANES)]
            cnt = (cnt[0] + jnp.sum((v == 0).astype(jnp.int32)),
                   cnt[1] + jnp.sum((v == 1).astype(jnp.int32)),
                   cnt[2] + jnp.sum((v == 2).astype(jnp.int32)),
                   cnt[3] + jnp.sum((v == 3).astype(jnp.int32)))
        cnt2[pl.ds(seg * LANES, LANES)] = (
            jnp.where(l16 == 0, cnt[0], 0) + jnp.where(l16 == 1, cnt[1], 0)
            + jnp.where(l16 == 2, cnt[2], 0) + jnp.where(l16 == 3, cnt[3], 0))
    pltpu.sync_copy(cnt2, cnt_sh.at[pl.ds(sid * 2 * LANES, 2 * LANES)])
    plsc.subcore_barrier()

    pltpu.sync_copy(cnt_sh, tab_v)
    total = jnp.zeros((LANES,), jnp.int32)
    prefix = jnp.zeros((LANES,), jnp.int32)
    for s in range(NSEG):
        row = tab_v[pl.ds(s * LANES, LANES)]
        total = total + row
        sv = jnp.full((LANES,), s, jnp.int32)
        prefix = prefix + jnp.where(sv < g, row, row * 0)
    padded = ((total + BLK - 1) >> 7) << 7
    ends = jnp.cumsum(padded)
    starts = ends - padded
    cursor = starts + prefix
    cur0 = jnp.sum(jnp.where(l16 == 0, cursor, 0))
    cur1 = jnp.sum(jnp.where(l16 == 1, cursor, 0))
    cur2 = jnp.sum(jnp.where(l16 == 2, cursor, 0))
    cur3 = jnp.sum(jnp.where(l16 == 3, cursor, 0))

    @pl.when((sid == 0) & (cid == 0))
    def _():
        e0 = jnp.sum(jnp.where(l16 == 0, ends, 0))
        e1 = jnp.sum(jnp.where(l16 == 1, ends, 0))
        e2 = jnp.sum(jnp.where(l16 == 2, ends, 0))
        e3 = jnp.sum(jnp.where(l16 == 3, ends, 0))
        for g16 in range(2):
            lo = (l16 + g16 * LANES) * BLK
            tbx = ((lo >= e0).astype(jnp.int32) + (lo >= e1).astype(jnp.int32)
                   + (lo >= e2).astype(jnp.int32) + (lo >= e3).astype(jnp.int32))
            tbv[pl.ds(g16 * LANES, LANES)] = jnp.minimum(tbx, NB - 1)
        pltpu.sync_copy(tbv, tb_hbm)

    pltpu.sync_copy(ids_hbm.at[pl.ds(g * SEG, SEG)], ids_v.at[pl.ds(0, SEG)])
    cur = (cur0, cur1, cur2, cur3)
    for ch in range(SEG // LANES):
        v = ids_v[pl.ds(ch * LANES, LANES)]
        m0 = (v == 0).astype(jnp.int32)
        m1 = (v == 1).astype(jnp.int32)
        m2 = (v == 2).astype(jnp.int32)
        m3 = (v == 3).astype(jnp.int32)
        base = m0 * cur[0] + m1 * cur[1] + m2 * cur[2] + m3 * cur[3]
        rank = (m0 * (jnp.cumsum(m0) - 1) + m1 * (jnp.cumsum(m1) - 1)
                + m2 * (jnp.cumsum(m2) - 1) + m3 * (jnp.cumsum(m3) - 1))
        slot_v[pl.ds(ch * LANES, LANES)] = base + rank
        cur = (cur[0] + jnp.sum(m0), cur[1] + jnp.sum(m1),
               cur[2] + jnp.sum(m2), cur[3] + jnp.sum(m3))

    pltpu.sync_copy(slot_v, inv_hbm.at[pl.ds(g * SEG, SEG)])
    pltpu.sync_copy(x_hbm.at[pl.ds(g * SEG, SEG)], rows_v)
    pltpu.async_copy(rows_v, xs_hbm.at[slot_v], sem).wait()


def _sc_sort_gather(ids, x2):
    mesh = plsc.VectorSubcoreMesh(core_axis_name="c", subcore_axis_name="s")
    return pl.kernel(
        _sc_sort_body,
        out_type=[
            jax.ShapeDtypeStruct((T_PAD, C), jnp.float32),
            jax.ShapeDtypeStruct((T,), jnp.int32),
            jax.ShapeDtypeStruct((NW,), jnp.int32),
        ],
        mesh=mesh,
        compiler_params=pltpu.CompilerParams(needs_layout_passes=False),
        scratch_types=[
            pltpu.VMEM((2 * SEG,), jnp.int32),
            pltpu.VMEM((2 * LANES,), jnp.int32),
            pltpu.VMEM((NSEG * LANES,), jnp.int32),
            pltpu.VMEM((SEG,), jnp.int32),
            pltpu.VMEM((NW,), jnp.int32),
            pltpu.VMEM((SEG, C), jnp.float32),
            pltpu.VMEM_SHARED((NSEG * LANES,), jnp.int32),
            pltpu.SemaphoreType.DMA,
        ],
    )(ids, x2)


def _sc_unsort_body(os_hbm, inv_hbm, out_hbm, inv_v, rows_v, sem):
    cid = lax.axis_index("c")
    sid = lax.axis_index("s")
    wid = cid * NS + sid
    base = wid * TPW
    pltpu.sync_copy(inv_hbm.at[pl.ds(base, TPW)], inv_v)
    pltpu.async_copy(os_hbm.at[inv_v], rows_v, sem).wait()
    pltpu.sync_copy(rows_v, out_hbm.at[pl.ds(base, TPW)])


def _sc_unsort(os, inv):
    mesh = plsc.VectorSubcoreMesh(core_axis_name="c", subcore_axis_name="s")
    return pl.kernel(
        _sc_unsort_body,
        out_type=jax.ShapeDtypeStruct((T, C), jnp.float32),
        mesh=mesh,
        compiler_params=pltpu.CompilerParams(needs_layout_passes=False),
        scratch_types=[
            pltpu.VMEM((TPW,), jnp.int32),
            pltpu.VMEM((TPW, C), jnp.float32),
            pltpu.SemaphoreType.DMA,
        ],
    )(os, inv)


def _moe_body(tb_ref, xs_ref, kp_ref, w1_ref, b1_ref, w2_ref, b2_ref,
              sw1_ref, sb1_ref, sw2_ref, sb2_ref, gate_ref, out_ref):
    xf = xs_ref[:]
    xv = xf.astype(jnp.bfloat16)
    kp = kp_ref[0]
    kn = kp / jnp.maximum(jnp.sqrt(jnp.sum(kp * kp, axis=1, keepdims=True)),
                          1e-12)
    d = (kn[1:2, :] - kn[0:1, :])
    nrm = jnp.maximum(jnp.sqrt(jnp.sum(xf * xf, axis=1, keepdims=True)), 1e-12)
    z = lax.dot_general(xf, d, (((1,), (1,)), ((), ())),
                        preferred_element_type=jnp.float32)
    p1 = jax.nn.sigmoid(z / nrm / max(TAU, 1e-6))
    alpha = jax.nn.sigmoid(gate_ref[0])
    h0 = jnp.maximum(
        jnp.dot(xv, w1_ref[0, 0], preferred_element_type=jnp.float32)
        + b1_ref[0, 0], 0.0).astype(jnp.bfloat16)
    h1 = jnp.maximum(
        jnp.dot(xv, w1_ref[0, 1], preferred_element_type=jnp.float32)
        + b1_ref[0, 1], 0.0).astype(jnp.bfloat16)
    y0 = jnp.dot(h0, w2_ref[0, 0], preferred_element_type=jnp.float32) + b2_ref[0, 0]
    y1 = jnp.dot(h1, w2_ref[0, 1], preferred_element_type=jnp.float32) + b2_ref[0, 1]
    hs = jnp.maximum(
        jnp.dot(xv, sw1_ref[:], preferred_element_type=jnp.float32)
        + sb1_ref[:], 0.0).astype(jnp.bfloat16)
    dense = jnp.dot(hs, sw2_ref[:], preferred_element_type=jnp.float32) + sb2_ref[:]
    out_ref[:] = dense + alpha * ((1.0 - p1) * y0 + p1 * y1)


def _moe_ffn(tb, xs, kpair, w1p, b1p, w2p, b2p, sW1, sb1, sW2, sb2, gate):
    grid_spec = pltpu.PrefetchScalarGridSpec(
        num_scalar_prefetch=1,
        grid=(G,),
        in_specs=[
            pl.BlockSpec((BLK, C), lambda i, tb: (i, 0)),
            pl.BlockSpec((1, EPB, C), lambda i, tb: (tb[i], 0, 0)),
            pl.BlockSpec((1, EPB, C, H), lambda i, tb: (tb[i], 0, 0, 0)),
            pl.BlockSpec((1, EPB, H), lambda i, tb: (tb[i], 0, 0)),
            pl.BlockSpec((1, EPB, H, C), lambda i, tb: (tb[i], 0, 0, 0)),
            pl.BlockSpec((1, EPB, C), lambda i, tb: (tb[i], 0, 0)),
            pl.BlockSpec((C, H), lambda i, tb: (0, 0)),
            pl.BlockSpec((1, H), lambda i, tb: (0, 0)),
            pl.BlockSpec((H, C), lambda i, tb: (0, 0)),
            pl.BlockSpec((1, C), lambda i, tb: (0, 0)),
            pl.BlockSpec(memory_space=pltpu.SMEM),
        ],
        out_specs=pl.BlockSpec((BLK, C), lambda i, tb: (i, 0)),
    )
    return pl.pallas_call(
        _moe_body,
        grid_spec=grid_spec,
        out_shape=jax.ShapeDtypeStruct((T_PAD, C), jnp.float32),
    )(tb, xs, kpair, w1p, b1p, w2p, b2p, sW1, sb1, sW2, sb2, gate)


def kernel(x, op_id, expert_key, sW1, sb1, sW2, sb2, eW1, eb1, eW2, eb2,
           gate_logit):
    x2 = x.reshape(T, C)
    ids = jnp.clip(op_id.reshape(T).astype(jnp.int32), 0, NB - 1)
    kpair = expert_key.reshape(NB, EPB, C)
    gate = gate_logit.reshape(1)

    xs, inv, tb = _sc_sort_gather(ids, x2)

    w1p = eW1.reshape(NB, EPB, C, H).astype(jnp.bfloat16)
    b1p = eb1.reshape(NB, EPB, H)
    w2p = eW2.reshape(NB, EPB, H, C).astype(jnp.bfloat16)
    b2p = eb2.reshape(NB, EPB, C)

    os = _moe_ffn(tb, xs, kpair, w1p, b1p, w2p, b2p,
                  sW1.astype(jnp.bfloat16), sb1.reshape(1, H),
                  sW2.astype(jnp.bfloat16), sb2.reshape(1, C), gate)

    out = _sc_unsort(os, inv)
    return out.reshape(x.shape)


# speedup vs baseline: 2.0728x; 1.0983x over previous
import functools

import jax
import jax.numpy as jnp
from jax import lax
from jax.experimental import pallas as pl
from jax.experimental.pallas import tpu as pltpu
from jax.experimental.pallas import tpu_sc as plsc

T, C, H = 2048, 768, 3072
NB, EPB = 4, 2
E = NB * EPB
TAU = 1.0
BLK = 256
BLK_SHIFT = 8
G = T // BLK + NB
T_PAD = G * BLK

NC, NS, LANES = 2, 16, 16
NW = NC * NS
RPW = T_PAD // NW
TPW = T // NW


NSEG = NW
SEG = T // NSEG


def _sc_sort_body(ids_hbm, x_hbm, xs_hbm, inv_hbm, tb_hbm,
                  ids_v, cnt2, tab_v, slot_v, tbv, rows_v, cnt_sh, sem):
    cid = lax.axis_index("c")
    sid = lax.axis_index("s")
    g = cid * NS + sid
    l16 = lax.iota(jnp.int32, 16)
    z = jnp.int32(0)

    pltpu.sync_copy(ids_hbm.at[pl.ds(sid * 2 * SEG, 2 * SEG)], ids_v)
    for seg in range(2):
        cnt = (z, z, z, z)
        for ch in range(SEG // LANES):
            v = ids_v[pl.ds((seg * SEG // LANES + ch) * LANES, LANES)]
            cnt = (cnt[0] + jnp.sum((v == 0).astype(jnp.int32)),
                   cnt[1] + jnp.sum((v == 1).astype(jnp.int32)),
                   cnt[2] + jnp.sum((v == 2).astype(jnp.int32)),
                   cnt[3] + jnp.sum((v == 3).astype(jnp.int32)))
        cnt2[pl.ds(seg * LANES, LANES)] = (
            jnp.where(l16 == 0, cnt[0], 0) + jnp.where(l16 == 1, cnt[1], 0)
            + jnp.where(l16 == 2, cnt[2], 0) + jnp.where(l16 == 3, cnt[3], 0))
    pltpu.sync_copy(cnt2, cnt_sh.at[pl.ds(sid * 2 * LANES, 2 * LANES)])
    plsc.subcore_barrier()

    pltpu.sync_copy(cnt_sh, tab_v)
    total = jnp.zeros((LANES,), jnp.int32)
    prefix = jnp.zeros((LANES,), jnp.int32)
    for s in range(NSEG):
        row = tab_v[pl.ds(s * LANES, LANES)]
        total = total + row
        sv = jnp.full((LANES,), s, jnp.int32)
        prefix = prefix + jnp.where(sv < g, row, row * 0)
    padded = ((total + BLK - 1) >> BLK_SHIFT) << BLK_SHIFT
    ends = jnp.cumsum(padded)
    starts = ends - padded
    cursor = starts + prefix
    cur0 = jnp.sum(jnp.where(l16 == 0, cursor, 0))
    cur1 = jnp.sum(jnp.where(l16 == 1, cursor, 0))
    cur2 = jnp.sum(jnp.where(l16 == 2, cursor, 0))
    cur3 = jnp.sum(jnp.where(l16 == 3, cursor, 0))

    @pl.when((sid == 0) & (cid == 0))
    def _():
        e0 = jnp.sum(jnp.where(l16 == 0, ends, 0))
        e1 = jnp.sum(jnp.where(l16 == 1, ends, 0))
        e2 = jnp.sum(jnp.where(l16 == 2, ends, 0))
        e3 = jnp.sum(jnp.where(l16 == 3, ends, 0))
        n_used = e3 >> BLK_SHIFT
        for g16 in range(2):
            lo = (l16 + g16 * LANES) * BLK
            tbx = ((lo >= e0).astype(jnp.int32) + (lo >= e1).astype(jnp.int32)
                   + (lo >= e2).astype(jnp.int32) + (lo >= e3).astype(jnp.int32))
            tbx = jnp.minimum(tbx, NB - 1)
            if g16 == 1:
                tbx = jnp.where(l16 == LANES - 1, n_used, tbx)
            tbv[pl.ds(g16 * LANES, LANES)] = tbx
        pltpu.sync_copy(tbv, tb_hbm)

    pltpu.sync_copy(ids_hbm.at[pl.ds(g * SEG, SEG)], ids_v.at[pl.ds(0, SEG)])
    cur = (cur0, cur1, cur2, cur3)
    for ch in range(SEG // LANES):
        v = ids_v[pl.ds(ch * LANES, LANES)]
        m0 = (v == 0).astype(jnp.int32)
        m1 = (v == 1).astype(jnp.int32)
        m2 = (v == 2).astype(jnp.int32)
        m3 = (v == 3).astype(jnp.int32)
        base = m0 * cur[0] + m1 * cur[1] + m2 * cur[2] + m3 * cur[3]
        rank = (m0 * (jnp.cumsum(m0) - 1) + m1 * (jnp.cumsum(m1) - 1)
                + m2 * (jnp.cumsum(m2) - 1) + m3 * (jnp.cumsum(m3) - 1))
        slot_v[pl.ds(ch * LANES, LANES)] = base + rank
        cur = (cur[0] + jnp.sum(m0), cur[1] + jnp.sum(m1),
               cur[2] + jnp.sum(m2), cur[3] + jnp.sum(m3))

    pltpu.sync_copy(slot_v, inv_hbm.at[pl.ds(g * SEG, SEG)])
    pltpu.sync_copy(x_hbm.at[pl.ds(g * SEG, SEG)], rows_v)
    pltpu.async_copy(rows_v, xs_hbm.at[slot_v], sem).wait()


def _sc_sort_gather(ids, x2):
    mesh = plsc.VectorSubcoreMesh(core_axis_name="c", subcore_axis_name="s")
    return pl.kernel(
        _sc_sort_body,
        out_type=[
            jax.ShapeDtypeStruct((T_PAD, C), jnp.float32),
            jax.ShapeDtypeStruct((T,), jnp.int32),
            jax.ShapeDtypeStruct((NW,), jnp.int32),
        ],
        mesh=mesh,
        compiler_params=pltpu.CompilerParams(needs_layout_passes=False),
        scratch_types=[
            pltpu.VMEM((2 * SEG,), jnp.int32),
            pltpu.VMEM((2 * LANES,), jnp.int32),
            pltpu.VMEM((NSEG * LANES,), jnp.int32),
            pltpu.VMEM((SEG,), jnp.int32),
            pltpu.VMEM((NW,), jnp.int32),
            pltpu.VMEM((SEG, C), jnp.float32),
            pltpu.VMEM_SHARED((NSEG * LANES,), jnp.int32),
            pltpu.SemaphoreType.DMA,
        ],
    )(ids, x2)


def _sc_unsort_body(os_hbm, inv_hbm, out_hbm, inv_v, rows_v, sem):
    cid = lax.axis_index("c")
    sid = lax.axis_index("s")
    wid = cid * NS + sid
    base = wid * TPW
    pltpu.sync_copy(inv_hbm.at[pl.ds(base, TPW)], inv_v)
    pltpu.async_copy(os_hbm.at[inv_v], rows_v, sem).wait()
    pltpu.sync_copy(rows_v, out_hbm.at[pl.ds(base, TPW)])


def _sc_unsort(os, inv):
    mesh = plsc.VectorSubcoreMesh(core_axis_name="c", subcore_axis_name="s")
    return pl.kernel(
        _sc_unsort_body,
        out_type=jax.ShapeDtypeStruct((T, C), jnp.float32),
        mesh=mesh,
        compiler_params=pltpu.CompilerParams(needs_layout_passes=False),
        scratch_types=[
            pltpu.VMEM((TPW,), jnp.int32),
            pltpu.VMEM((TPW, C), jnp.float32),
            pltpu.SemaphoreType.DMA,
        ],
    )(os, inv)


def _moe_body(tb_ref, xs_ref, kp_ref, w1_ref, b1_ref, w2_ref, b2_ref,
              sw1_ref, sb1_ref, sw2_ref, sb2_ref, gate_ref, out_ref):
    i = pl.program_id(0)

    @pl.when(i < tb_ref[NW - 1])
    def _():
        _moe_tile(xs_ref, kp_ref, w1_ref, b1_ref, w2_ref, b2_ref,
                  sw1_ref, sb1_ref, sw2_ref, sb2_ref, gate_ref, out_ref)


def _moe_tile(xs_ref, kp_ref, w1_ref, b1_ref, w2_ref, b2_ref,
              sw1_ref, sb1_ref, sw2_ref, sb2_ref, gate_ref, out_ref):
    xf = xs_ref[:]
    xv = xf.astype(jnp.bfloat16)
    kp = kp_ref[0]
    kn = kp / jnp.maximum(jnp.sqrt(jnp.sum(kp * kp, axis=1, keepdims=True)),
                          1e-12)
    d = (kn[1:2, :] - kn[0:1, :])
    nrm = jnp.maximum(jnp.sqrt(jnp.sum(xf * xf, axis=1, keepdims=True)), 1e-12)
    z = lax.dot_general(xf, d, (((1,), (1,)), ((), ())),
                        preferred_element_type=jnp.float32)
    p1 = jax.nn.sigmoid(z / nrm / max(TAU, 1e-6))
    alpha = jax.nn.sigmoid(gate_ref[0])
    h0 = jnp.maximum(
        jnp.dot(xv, w1_ref[0, 0], preferred_element_type=jnp.float32)
        + b1_ref[0, 0], 0.0).astype(jnp.bfloat16)
    h1 = jnp.maximum(
        jnp.dot(xv, w1_ref[0, 1], preferred_element_type=jnp.float32)
        + b1_ref[0, 1], 0.0).astype(jnp.bfloat16)
    y0 = jnp.dot(h0, w2_ref[0, 0], preferred_element_type=jnp.float32) + b2_ref[0, 0]
    y1 = jnp.dot(h1, w2_ref[0, 1], preferred_element_type=jnp.float32) + b2_ref[0, 1]
    hs = jnp.maximum(
        jnp.dot(xv, sw1_ref[:], preferred_element_type=jnp.float32)
        + sb1_ref[:], 0.0).astype(jnp.bfloat16)
    dense = jnp.dot(hs, sw2_ref[:], preferred_element_type=jnp.float32) + sb2_ref[:]
    out_ref[:] = dense + alpha * ((1.0 - p1) * y0 + p1 * y1)


def _moe_ffn(tb, xs, kpair, w1p, b1p, w2p, b2p, sW1, sb1, sW2, sb2, gate):
    grid_spec = pltpu.PrefetchScalarGridSpec(
        num_scalar_prefetch=1,
        grid=(G,),
        in_specs=[
            pl.BlockSpec((BLK, C), lambda i, tb: (i, 0)),
            pl.BlockSpec((1, EPB, C), lambda i, tb: (tb[i], 0, 0)),
            pl.BlockSpec((1, EPB, C, H), lambda i, tb: (tb[i], 0, 0, 0)),
            pl.BlockSpec((1, EPB, H), lambda i, tb: (tb[i], 0, 0)),
            pl.BlockSpec((1, EPB, H, C), lambda i, tb: (tb[i], 0, 0, 0)),
            pl.BlockSpec((1, EPB, C), lambda i, tb: (tb[i], 0, 0)),
            pl.BlockSpec((C, H), lambda i, tb: (0, 0)),
            pl.BlockSpec((1, H), lambda i, tb: (0, 0)),
            pl.BlockSpec((H, C), lambda i, tb: (0, 0)),
            pl.BlockSpec((1, C), lambda i, tb: (0, 0)),
            pl.BlockSpec(memory_space=pltpu.SMEM),
        ],
        out_specs=pl.BlockSpec((BLK, C), lambda i, tb: (i, 0)),
    )
    return pl.pallas_call(
        _moe_body,
        grid_spec=grid_spec,
        out_shape=jax.ShapeDtypeStruct((T_PAD, C), jnp.float32),
    )(tb, xs, kpair, w1p, b1p, w2p, b2p, sW1, sb1, sW2, sb2, gate)


def kernel(x, op_id, expert_key, sW1, sb1, sW2, sb2, eW1, eb1, eW2, eb2,
           gate_logit):
    x2 = x.reshape(T, C)
    ids = jnp.clip(op_id.reshape(T).astype(jnp.int32), 0, NB - 1)
    kpair = expert_key.reshape(NB, EPB, C)
    gate = gate_logit.reshape(1)

    xs, inv, tb = _sc_sort_gather(ids, x2)

    w1p = eW1.reshape(NB, EPB, C, H).astype(jnp.bfloat16)
    b1p = eb1.reshape(NB, EPB, H)
    w2p = eW2.reshape(NB, EPB, H, C).astype(jnp.bfloat16)
    b2p = eb2.reshape(NB, EPB, C)

    os = _moe_ffn(tb, xs, kpair, w1p, b1p, w2p, b2p,
                  sW1.astype(jnp.bfloat16), sb1.reshape(1, H),
                  sW2.astype(jnp.bfloat16), sb2.reshape(1, C), gate)

    out = _sc_unsort(os, inv)
    return out.reshape(x.shape)


# speedup vs baseline: 2.3695x; 1.1432x over previous
import functools

import jax
import jax.numpy as jnp
from jax import lax
from jax.experimental import pallas as pl
from jax.experimental.pallas import tpu as pltpu
from jax.experimental.pallas import tpu_sc as plsc

T, C, H = 2048, 768, 3072
NB, EPB = 4, 2
E = NB * EPB
TAU = 1.0
BLK = 256
BLK_SHIFT = 8
G = T // BLK + NB
T_PAD = G * BLK

NC, NS, LANES = 2, 16, 16
NW = NC * NS
RPW = T_PAD // NW
TPW = T // NW


NSEG = NW
SEG = T // NSEG


def _sc_sort_body(ids_hbm, x_hbm, xs_hbm, inv_hbm, tb_hbm,
                  ids_v, cnt2, tab_v, slot_v, tbv, rows_v, cnt_sh, sem):
    cid = lax.axis_index("c")
    sid = lax.axis_index("s")
    g = cid * NS + sid
    l16 = lax.iota(jnp.int32, 16)
    z = jnp.int32(0)

    pltpu.sync_copy(ids_hbm.at[pl.ds(sid * 2 * SEG, 2 * SEG)], ids_v)
    for seg in range(2):
        cnt = (z, z, z, z)
        for ch in range(SEG // LANES):
            v = ids_v[pl.ds((seg * SEG // LANES + ch) * LANES, LANES)]
            cnt = (cnt[0] + jnp.sum((v == 0).astype(jnp.int32)),
                   cnt[1] + jnp.sum((v == 1).astype(jnp.int32)),
                   cnt[2] + jnp.sum((v == 2).astype(jnp.int32)),
                   cnt[3] + jnp.sum((v == 3).astype(jnp.int32)))
        cnt2[pl.ds(seg * LANES, LANES)] = (
            jnp.where(l16 == 0, cnt[0], 0) + jnp.where(l16 == 1, cnt[1], 0)
            + jnp.where(l16 == 2, cnt[2], 0) + jnp.where(l16 == 3, cnt[3], 0))
    pltpu.sync_copy(cnt2, cnt_sh.at[pl.ds(sid * 2 * LANES, 2 * LANES)])
    plsc.subcore_barrier()

    pltpu.sync_copy(cnt_sh, tab_v)
    total = jnp.zeros((LANES,), jnp.int32)
    prefix = jnp.zeros((LANES,), jnp.int32)
    for s in range(NSEG):
        row = tab_v[pl.ds(s * LANES, LANES)]
        total = total + row
        sv = jnp.full((LANES,), s, jnp.int32)
        prefix = prefix + jnp.where(sv < g, row, row * 0)
    padded = ((total + BLK - 1) >> BLK_SHIFT) << BLK_SHIFT
    ends = jnp.cumsum(padded)
    starts = ends - padded
    cursor = starts + prefix
    cur0 = jnp.sum(jnp.where(l16 == 0, cursor, 0))
    cur1 = jnp.sum(jnp.where(l16 == 1, cursor, 0))
    cur2 = jnp.sum(jnp.where(l16 == 2, cursor, 0))
    cur3 = jnp.sum(jnp.where(l16 == 3, cursor, 0))

    @pl.when((sid == 0) & (cid == 0))
    def _():
        e0 = jnp.sum(jnp.where(l16 == 0, ends, 0))
        e1 = jnp.sum(jnp.where(l16 == 1, ends, 0))
        e2 = jnp.sum(jnp.where(l16 == 2, ends, 0))
        e3 = jnp.sum(jnp.where(l16 == 3, ends, 0))
        n_used = e3 >> BLK_SHIFT
        for g16 in range(2):
            lo = (l16 + g16 * LANES) * BLK
            tbx = ((lo >= e0).astype(jnp.int32) + (lo >= e1).astype(jnp.int32)
                   + (lo >= e2).astype(jnp.int32) + (lo >= e3).astype(jnp.int32))
            tbx = jnp.minimum(tbx, NB - 1)
            if g16 == 1:
                tbx = jnp.where(l16 == LANES - 1, n_used, tbx)
            tbv[pl.ds(g16 * LANES, LANES)] = tbx
        pltpu.sync_copy(tbv, tb_hbm)

    pltpu.sync_copy(ids_hbm.at[pl.ds(g * SEG, SEG)], ids_v.at[pl.ds(0, SEG)])
    cur = (cur0, cur1, cur2, cur3)
    for ch in range(SEG // LANES):
        v = ids_v[pl.ds(ch * LANES, LANES)]
        m0 = (v == 0).astype(jnp.int32)
        m1 = (v == 1).astype(jnp.int32)
        m2 = (v == 2).astype(jnp.int32)
        m3 = (v == 3).astype(jnp.int32)
        base = m0 * cur[0] + m1 * cur[1] + m2 * cur[2] + m3 * cur[3]
        rank = (m0 * (jnp.cumsum(m0) - 1) + m1 * (jnp.cumsum(m1) - 1)
                + m2 * (jnp.cumsum(m2) - 1) + m3 * (jnp.cumsum(m3) - 1))
        slot_v[pl.ds(ch * LANES, LANES)] = base + rank
        cur = (cur[0] + jnp.sum(m0), cur[1] + jnp.sum(m1),
               cur[2] + jnp.sum(m2), cur[3] + jnp.sum(m3))

    pltpu.sync_copy(slot_v, inv_hbm.at[pl.ds(g * SEG, SEG)])
    pltpu.sync_copy(x_hbm.at[pl.ds(g * SEG, SEG)], rows_v)
    pltpu.async_copy(rows_v, xs_hbm.at[slot_v], sem).wait()


def _sc_sort_gather(ids, x2):
    mesh = plsc.VectorSubcoreMesh(core_axis_name="c", subcore_axis_name="s")
    return pl.kernel(
        _sc_sort_body,
        out_type=[
            jax.ShapeDtypeStruct((T_PAD, C), jnp.float32),
            jax.ShapeDtypeStruct((T,), jnp.int32),
            jax.ShapeDtypeStruct((NW,), jnp.int32),
        ],
        mesh=mesh,
        compiler_params=pltpu.CompilerParams(needs_layout_passes=False),
        scratch_types=[
            pltpu.VMEM((2 * SEG,), jnp.int32),
            pltpu.VMEM((2 * LANES,), jnp.int32),
            pltpu.VMEM((NSEG * LANES,), jnp.int32),
            pltpu.VMEM((SEG,), jnp.int32),
            pltpu.VMEM((NW,), jnp.int32),
            pltpu.VMEM((SEG, C), jnp.float32),
            pltpu.VMEM_SHARED((NSEG * LANES,), jnp.int32),
            pltpu.SemaphoreType.DMA,
        ],
    )(ids, x2)


def _sc_unsort_body(os_hbm, inv_hbm, out_hbm, inv_v, rows_v, sem):
    cid = lax.axis_index("c")
    sid = lax.axis_index("s")
    wid = cid * NS + sid
    base = wid * TPW
    pltpu.sync_copy(inv_hbm.at[pl.ds(base, TPW)], inv_v)
    pltpu.async_copy(os_hbm.at[inv_v], rows_v, sem).wait()
    pltpu.sync_copy(rows_v, out_hbm.at[pl.ds(base, TPW)])


def _sc_unsort(os, inv):
    mesh = plsc.VectorSubcoreMesh(core_axis_name="c", subcore_axis_name="s")
    return pl.kernel(
        _sc_unsort_body,
        out_type=jax.ShapeDtypeStruct((T, C), jnp.float32),
        mesh=mesh,
        compiler_params=pltpu.CompilerParams(needs_layout_passes=False),
        scratch_types=[
            pltpu.VMEM((TPW,), jnp.int32),
            pltpu.VMEM((TPW, C), jnp.float32),
            pltpu.SemaphoreType.DMA,
        ],
    )(os, inv)


NH = 3
HB = H // NH


def _moe_body(tb_ref, xs_ref, kp_ref, w1_ref, b1_ref, w2_ref, b2_ref,
              sw1_ref, sb1_ref, sw2_ref, sb2_ref, gate_ref, out_ref, acc_ref):
    j = pl.program_id(0)
    i = pl.program_id(1)

    @pl.when(i < tb_ref[NW - 1])
    def _():
        xf = xs_ref[:]
        xv = xf.astype(jnp.bfloat16)
        kp = kp_ref[0]
        kn = kp / jnp.maximum(
            jnp.sqrt(jnp.sum(kp * kp, axis=1, keepdims=True)), 1e-12)
        d = (kn[1:2, :] - kn[0:1, :])
        nrm = jnp.maximum(jnp.sqrt(jnp.sum(xf * xf, axis=1, keepdims=True)),
                          1e-12)
        z = lax.dot_general(xf, d, (((1,), (1,)), ((), ())),
                            preferred_element_type=jnp.float32)
        p1 = jax.nn.sigmoid(z / nrm / max(TAU, 1e-6))
        alpha = jax.nn.sigmoid(gate_ref[0])
        w1b = w1_ref[0].astype(jnp.bfloat16)
        w2b = w2_ref[0].astype(jnp.bfloat16)
        h0 = jnp.maximum(
            jnp.dot(xv, w1b[0], preferred_element_type=jnp.float32)
            + b1_ref[0, 0], 0.0).astype(jnp.bfloat16)
        h1 = jnp.maximum(
            jnp.dot(xv, w1b[1], preferred_element_type=jnp.float32)
            + b1_ref[0, 1], 0.0).astype(jnp.bfloat16)
        y0 = jnp.dot(h0, w2b[0], preferred_element_type=jnp.float32)
        y1 = jnp.dot(h1, w2b[1], preferred_element_type=jnp.float32)
        hs = jnp.maximum(
            jnp.dot(xv, sw1_ref[:], preferred_element_type=jnp.float32)
            + sb1_ref[:], 0.0).astype(jnp.bfloat16)
        part = (jnp.dot(hs, sw2_ref[:], preferred_element_type=jnp.float32)
                + alpha * ((1.0 - p1) * y0 + p1 * y1))

        @pl.when(j == 0)
        def _():
            acc_ref[i] = (part + sb2_ref[:]
                          + alpha * ((1.0 - p1) * b2_ref[0, 0]
                                     + p1 * b2_ref[0, 1]))

        @pl.when(j > 0)
        def _():
            tot = acc_ref[i] + part
            acc_ref[i] = tot
            out_ref[:] = tot


def _moe_ffn(tb, xs, kpair, w1p, b1p, w2p, b2p, sW1, sb1, sW2, sb2, gate):
    grid_spec = pltpu.PrefetchScalarGridSpec(
        num_scalar_prefetch=1,
        grid=(NH, G),
        in_specs=[
            pl.BlockSpec((BLK, C), lambda j, i, tb: (i, 0)),
            pl.BlockSpec((1, EPB, C), lambda j, i, tb: (tb[i], 0, 0)),
            pl.BlockSpec((1, EPB, C, HB), lambda j, i, tb: (tb[i], 0, 0, j)),
            pl.BlockSpec((1, EPB, HB), lambda j, i, tb: (tb[i], 0, j)),
            pl.BlockSpec((1, EPB, HB, C), lambda j, i, tb: (tb[i], 0, j, 0)),
            pl.BlockSpec((1, EPB, C), lambda j, i, tb: (tb[i], 0, 0)),
            pl.BlockSpec((C, HB), lambda j, i, tb: (0, j)),
            pl.BlockSpec((1, HB), lambda j, i, tb: (0, j)),
            pl.BlockSpec((HB, C), lambda j, i, tb: (j, 0)),
            pl.BlockSpec((1, C), lambda j, i, tb: (0, 0)),
            pl.BlockSpec(memory_space=pltpu.SMEM),
        ],
        out_specs=pl.BlockSpec((BLK, C), lambda j, i, tb: (i, 0)),
        scratch_shapes=[pltpu.VMEM((G, BLK, C), jnp.float32)],
    )
    return pl.pallas_call(
        _moe_body,
        grid_spec=grid_spec,
        compiler_params=pltpu.CompilerParams(
            vmem_limit_bytes=63 * 1024 * 1024),
        out_shape=jax.ShapeDtypeStruct((T_PAD, C), jnp.float32),
    )(tb, xs, kpair, w1p, b1p, w2p, b2p, sW1, sb1, sW2, sb2, gate)


def kernel(x, op_id, expert_key, sW1, sb1, sW2, sb2, eW1, eb1, eW2, eb2,
           gate_logit):
    x2 = x.reshape(T, C)
    ids = jnp.clip(op_id.reshape(T).astype(jnp.int32), 0, NB - 1)
    kpair = expert_key.reshape(NB, EPB, C)
    gate = gate_logit.reshape(1)

    xs, inv, tb = _sc_sort_gather(ids, x2)

    w1p = eW1.reshape(NB, EPB, C, H)
    b1p = eb1.reshape(NB, EPB, H)
    w2p = eW2.reshape(NB, EPB, H, C)
    b2p = eb2.reshape(NB, EPB, C)

    os = _moe_ffn(tb, xs, kpair, w1p, b1p, w2p, b2p,
                  sW1.astype(jnp.bfloat16), sb1.reshape(1, H),
                  sW2.astype(jnp.bfloat16), sb2.reshape(1, C), gate)

    out = _sc_unsort(os, inv)
    return out.reshape(x.shape)
